# baseline (speedup 1.0000x reference)
import functools

import jax
import jax.numpy as jnp
from jax import lax
from jax.experimental import pallas as pl
from jax.experimental.pallas import tpu as pltpu
from jax.experimental.pallas import tpu_sc as plsc

_N = 10000
_E = 320000
_IN = 128
_HID = 256
_OUT = 128

_NC = 2
_NS = 16
_NW = _NC * _NS
_CHUNK = 96
_ET = _E // _NW
_NCHUNK = -(-_ET // _CHUNK)
_ETP = _NCHUNK * _CHUNK
_PAD = _ETP - _ET
_NP = 10112
_ROWS_T = _NP // _NS
_DEG_W = 128
_DEG_WIN = 8


def _sc_mesh():
    return plsc.VectorSubcoreMesh(core_axis_name="c", subcore_axis_name="s")


def _deg_call(dst2d, ones_c, zeros_c):
    @functools.partial(
        pl.kernel,
        out_type=jax.ShapeDtypeStruct((_NC, _NP, _DEG_W), jnp.float32),
        mesh=_sc_mesh(),
        scratch_types=[
            pltpu.VMEM((_NCHUNK, _CHUNK), jnp.int32),
            pltpu.VMEM((_CHUNK, _DEG_W), jnp.float32),
            pltpu.VMEM((128, _DEG_W), jnp.float32),
            pltpu.VMEM_SHARED((_NP, _DEG_W), jnp.float32),
            pltpu.SemaphoreType.DMA,
        ],
    )
    def k(dst_hbm, ones_hbm, zeros_hbm, out_hbm, idx_v, ones_v, zb_v, acc_sh, sem):
        c = lax.axis_index("c")
        s = lax.axis_index("s")
        wid = c * _NS + s
        pltpu.sync_copy(dst_hbm.at[wid], idx_v)
        pltpu.sync_copy(ones_hbm, ones_v)
        pltpu.sync_copy(zeros_hbm, zb_v)
        for j in range(_ROWS_T // 128):
            pltpu.sync_copy(zb_v, acc_sh.at[pl.ds(s * _ROWS_T + j * 128, 128)])
        pltpu.sync_copy(
            zb_v.at[pl.ds(0, _ROWS_T % 128)],
            acc_sh.at[pl.ds(s * _ROWS_T + (_ROWS_T // 128) * 128, _ROWS_T % 128)],
        )
        plsc.subcore_barrier()

        for i in range(_DEG_WIN):
            pltpu.async_copy(ones_v, acc_sh.at[idx_v.at[i]], sem, add=True)

        def body(i, _):
            pltpu.make_async_copy(ones_v, acc_sh.at[idx_v.at[i]], sem).wait()
            pltpu.async_copy(ones_v, acc_sh.at[idx_v.at[i + _DEG_WIN]], sem, add=True)
            return 0

        lax.fori_loop(0, _NCHUNK - _DEG_WIN, body, 0)
        for i in range(_DEG_WIN):
            pltpu.make_async_copy(ones_v, acc_sh.at[idx_v.at[i]], sem).wait()
        plsc.subcore_barrier()
        pltpu.sync_copy(
            acc_sh.at[pl.ds(s * _ROWS_T, _ROWS_T)],
            out_hbm.at[c, pl.ds(s * _ROWS_T, _ROWS_T)],
        )

    return k(dst2d, ones_c, zeros_c)


def _make_scatter(n_tables):
    @functools.partial(
        pl.kernel,
        out_type=jax.ShapeDtypeStruct((n_tables, _NC, _NP, 128), jnp.float32),
        mesh=_sc_mesh(),
        scratch_types=[
            pltpu.VMEM((_ETP,), jnp.int32),
            pltpu.VMEM((_NCHUNK, _CHUNK), jnp.int32),
            pltpu.VMEM((_CHUNK, 128), jnp.float32),
            pltpu.VMEM((_CHUNK, 128), jnp.float32),
            pltpu.VMEM_SHARED((_NP, 128), jnp.float32),
            pltpu.SemaphoreType.DMA,
            pltpu.SemaphoreType.DMA,
            pltpu.SemaphoreType.DMA,
            pltpu.SemaphoreType.DMA,
        ],
    )
    def k(*refs):
        tables = refs[:n_tables]
        (src_hbm, dst_hbm, out_hbm, isrc_v, idst_v, rows_a, rows_b, acc_sh,
         sga, sgb, ssa, ssb) = refs[n_tables:]
        c = lax.axis_index("c")
        s = lax.axis_index("s")
        wid = c * _NS + s
        pltpu.sync_copy(src_hbm.at[pl.ds(wid * _ETP, _ETP)], isrc_v)
        pltpu.sync_copy(dst_hbm.at[wid], idst_v)

        def zfill(i, _):
            for j in range(8):
                rows_a[i, pl.ds(16 * j, 16)] = jnp.zeros((16,), jnp.float32)
            return 0

        for t in range(n_tables):
            tbl = tables[t]
            lax.fori_loop(0, _CHUNK, zfill, 0)
            for j in range(_ROWS_T // _CHUNK):
                pltpu.sync_copy(rows_a, acc_sh.at[pl.ds(s * _ROWS_T + j * _CHUNK, _CHUNK)])
            pltpu.sync_copy(
                rows_a.at[pl.ds(0, _ROWS_T % _CHUNK)],
                acc_sh.at[pl.ds(s * _ROWS_T + (_ROWS_T // _CHUNK) * _CHUNK, _ROWS_T % _CHUNK)],
            )
            plsc.subcore_barrier()

            pltpu.async_copy(tbl.at[isrc_v.at[pl.ds(0, _CHUNK)]], rows_a, sga)
            pltpu.async_copy(tbl.at[isrc_v.at[pl.ds(_CHUNK, _CHUNK)]], rows_b, sgb)

            def body(p, _):
                i0 = 2 * p
                i1 = i0 + 1
                pltpu.make_async_copy(tbl.at[isrc_v.at[pl.ds(i0 * _CHUNK, _CHUNK)]], rows_a, sga).wait()
                pltpu.async_copy(rows_a, acc_sh.at[idst_v.at[i0]], ssa, add=True)
                pltpu.make_async_copy(tbl.at[isrc_v.at[pl.ds(i1 * _CHUNK, _CHUNK)]], rows_b, sgb).wait()
                pltpu.async_copy(rows_b, acc_sh.at[idst_v.at[i1]], ssb, add=True)
                pltpu.make_async_copy(rows_a, acc_sh.at[idst_v.at[i0]], ssa).wait()

                @pl.when(i0 + 2 < _NCHUNK)
                def _():
                    pltpu.async_copy(tbl.at[isrc_v.at[pl.ds((i0 + 2) * _CHUNK, _CHUNK)]], rows_a, sga)

                pltpu.make_async_copy(rows_b, acc_sh.at[idst_v.at[i1]], ssb).wait()

                @pl.when(i1 + 2 < _NCHUNK)
                def _():
                    pltpu.async_copy(tbl.at[isrc_v.at[pl.ds((i1 + 2) * _CHUNK, _CHUNK)]], rows_b, sgb)

                return 0

            lax.fori_loop(0, _NCHUNK // 2, body, 0)
            if _NCHUNK % 2:
                ilast = _NCHUNK - 1
                pltpu.make_async_copy(tbl.at[isrc_v.at[pl.ds(ilast * _CHUNK, _CHUNK)]], rows_a, sga).wait()
                pltpu.async_copy(rows_a, acc_sh.at[idst_v.at[ilast]], ssa, add=True)
                pltpu.make_async_copy(rows_a, acc_sh.at[idst_v.at[ilast]], ssa).wait()
            plsc.subcore_barrier()
            pltpu.sync_copy(
                acc_sh.at[pl.ds(s * _ROWS_T, _ROWS_T)],
                out_hbm.at[t, c, pl.ds(s * _ROWS_T, _ROWS_T)],
            )
            if t + 1 < n_tables:
                plsc.subcore_barrier()

    return k


_scatter1 = _make_scatter(1)


_BLK = 2000
_GRID = _N // _BLK


def _tcA_call(x, deg_p):
    def body(x_ref, dp_ref, z_ref):
        deg = dp_ref[0] + dp_ref[1]
        dinv = lax.rsqrt(deg[:, :1] + 1.0)
        z_ref[:, :] = x_ref[:, :] * dinv

    return pl.pallas_call(
        body,
        grid=(_GRID,),
        in_specs=[
            pl.BlockSpec((_BLK, _IN), lambda i: (i, 0)),
            pl.BlockSpec((_NC, _BLK, _DEG_W), lambda i: (0, i, 0)),
        ],
        out_specs=pl.BlockSpec((_BLK, _IN), lambda i: (i, 0)),
        out_shape=jax.ShapeDtypeStruct((_N, _IN), jnp.float32),
    )(x, deg_p)


def _tcB_call(agg1, x, deg_p, b1_r, W1, W2):
    def body(a_ref, x_ref, dp_ref, b1_ref, w1_ref, w2_ref, y2_ref, s2_ref):
        deg = dp_ref[0] + dp_ref[1]
        dinv = lax.rsqrt(deg[:, :1] + 1.0)
        u = (a_ref[0, 0] + a_ref[0, 1]) * dinv + x_ref[:, :] * (dinv * dinv)
        h = jnp.dot(u, w1_ref[:, :], preferred_element_type=jnp.float32)
        h = jnp.maximum(h + b1_ref[0:1, :], 0.0)
        xw = jnp.dot(h, w2_ref[:, :], preferred_element_type=jnp.float32)
        y2_ref[:, :] = xw * dinv
        s2_ref[:, :] = xw * (dinv * dinv)

    return pl.pallas_call(
        body,
        grid=(_GRID,),
        in_specs=[
            pl.BlockSpec((1, _NC, _BLK, 128), lambda i: (0, 0, i, 0)),
            pl.BlockSpec((_BLK, _IN), lambda i: (i, 0)),
            pl.BlockSpec((_NC, _BLK, _DEG_W), lambda i: (0, i, 0)),
            pl.BlockSpec((1, _HID), lambda i: (0, 0)),
            pl.BlockSpec((_IN, _HID), lambda i: (0, 0)),
            pl.BlockSpec((_HID, _OUT), lambda i: (0, 0)),
        ],
        out_specs=[
            pl.BlockSpec((_BLK, _OUT), lambda i: (i, 0)),
            pl.BlockSpec((_BLK, _OUT), lambda i: (i, 0)),
        ],
        out_shape=[
            jax.ShapeDtypeStruct((_N, _OUT), jnp.float32),
            jax.ShapeDtypeStruct((_N, _OUT), jnp.float32),
        ],
    )(agg1, x, deg_p, b1_r, W1, W2)


def _tcC_call(acc2, self2, deg_p, b2_r):
    def body(a_ref, s2_ref, dp_ref, b2_ref, o_ref):
        deg = dp_ref[0] + dp_ref[1]
        dinv = lax.rsqrt(deg[:, :1] + 1.0)
        acc = a_ref[0, 0] + a_ref[0, 1]
        o_ref[:, :] = acc * dinv + s2_ref[:, :] + b2_ref[0:1, :]

    return pl.pallas_call(
        body,
        grid=(_GRID,),
        in_specs=[
            pl.BlockSpec((1, _NC, _BLK, 128), lambda i: (0, 0, i, 0)),
            pl.BlockSpec((_BLK, _OUT), lambda i: (i, 0)),
            pl.BlockSpec((_NC, _BLK, _DEG_W), lambda i: (0, i, 0)),
            pl.BlockSpec((1, 128), lambda i: (0, 0)),
        ],
        out_specs=pl.BlockSpec((_BLK, _OUT), lambda i: (i, 0)),
        out_shape=jax.ShapeDtypeStruct((_N, _OUT), jnp.float32),
    )(acc2, self2, deg_p, b2_r)


def kernel(x, edge_index, W1, b1, W2, b2):
    src_r = edge_index[0].astype(jnp.int32).reshape(_NW, _ET)
    dst_r = edge_index[1].astype(jnp.int32).reshape(_NW, _ET)
    src1d = jnp.pad(src_r, ((0, 0), (0, _PAD))).reshape(_NW * _ETP)
    pad_dst = jnp.broadcast_to(_N + (jnp.arange(_PAD, dtype=jnp.int32) % (_NP - _N)),
                               (_NW, _PAD))
    dst2d = jnp.concatenate([dst_r, pad_dst], axis=1).reshape(_NW, _NCHUNK, _CHUNK)
    b1_r = b1.reshape(1, _HID)
    b2_r = b2.reshape(1, 128)

    ones_c = jnp.ones((_CHUNK, _DEG_W), jnp.float32)
    zeros_c = jnp.zeros((128, _DEG_W), jnp.float32)
    deg_p = _deg_call(dst2d, ones_c, zeros_c)
    z = _tcA_call(x, deg_p)
    agg1 = _scatter1(z, src1d, dst2d)
    y2, self2 = _tcB_call(agg1, x, deg_p, b1_r, W1, W2)
    acc2 = _scatter1(y2, src1d, dst2d)
    out = _tcC_call(acc2, self2, deg_p, b2_r)
    return out


# speedup vs baseline: 1.3923x; 1.3923x over previous
import functools

import jax
import jax.numpy as jnp
from jax import lax
from jax.experimental import pallas as pl
from jax.experimental.pallas import tpu as pltpu
from jax.experimental.pallas import tpu_sc as plsc

_N = 10000
_E = 320000
_IN = 128
_HID = 256
_OUT = 128

_NC = 2
_NS = 16
_NW = _NC * _NS
_CHUNK = 80
_ET = _E // _NW
_NCHUNK = _ET // _CHUNK
_NP = 10112
_ROWS_T = _NP // _NS
_DEG_W = 128
_DEG_WIN = 8


def _sc_mesh():
    return plsc.VectorSubcoreMesh(core_axis_name="c", subcore_axis_name="s")


def _deg_call(dst2d, ones_c, zeros_c):
    @functools.partial(
        pl.kernel,
        out_type=jax.ShapeDtypeStruct((_NC, _NP, _DEG_W), jnp.float32),
        mesh=_sc_mesh(),
        scratch_types=[
            pltpu.VMEM((_NCHUNK, _CHUNK), jnp.int32),
            pltpu.VMEM((_CHUNK, _DEG_W), jnp.float32),
            pltpu.VMEM((128, _DEG_W), jnp.float32),
            pltpu.VMEM_SHARED((_NP, _DEG_W), jnp.float32),
            pltpu.SemaphoreType.DMA,
        ],
    )
    def k(dst_hbm, ones_hbm, zeros_hbm, out_hbm, idx_v, ones_v, zb_v, acc_sh, sem):
        c = lax.axis_index("c")
        s = lax.axis_index("s")
        wid = c * _NS + s
        pltpu.sync_copy(dst_hbm.at[wid], idx_v)
        pltpu.sync_copy(ones_hbm, ones_v)
        pltpu.sync_copy(zeros_hbm, zb_v)
        for j in range(_ROWS_T // 128):
            pltpu.sync_copy(zb_v, acc_sh.at[pl.ds(s * _ROWS_T + j * 128, 128)])
        pltpu.sync_copy(
            zb_v.at[pl.ds(0, _ROWS_T % 128)],
            acc_sh.at[pl.ds(s * _ROWS_T + (_ROWS_T // 128) * 128, _ROWS_T % 128)],
        )
        plsc.subcore_barrier()

        for i in range(_DEG_WIN):
            pltpu.async_copy(ones_v, acc_sh.at[idx_v.at[i]], sem, add=True)

        def body(i, _):
            pltpu.make_async_copy(ones_v, acc_sh.at[idx_v.at[i]], sem).wait()
            pltpu.async_copy(ones_v, acc_sh.at[idx_v.at[i + _DEG_WIN]], sem, add=True)
            return 0

        lax.fori_loop(0, _NCHUNK - _DEG_WIN, body, 0)
        for i in range(_DEG_WIN):
            pltpu.make_async_copy(ones_v, acc_sh.at[idx_v.at[i]], sem).wait()
        plsc.subcore_barrier()
        pltpu.sync_copy(
            acc_sh.at[pl.ds(s * _ROWS_T, _ROWS_T)],
            out_hbm.at[c, pl.ds(s * _ROWS_T, _ROWS_T)],
        )

    return k(dst2d, ones_c, zeros_c)


def _make_scatter(n_tables):
    @functools.partial(
        pl.kernel,
        out_type=jax.ShapeDtypeStruct((n_tables, _NC, _NP, 128), jnp.float32),
        mesh=_sc_mesh(),
        scratch_types=[
            pltpu.VMEM((_ET,), jnp.int32),
            pltpu.VMEM((_NCHUNK, _CHUNK), jnp.int32),
            pltpu.VMEM((_CHUNK, 128), jnp.float32),
            pltpu.VMEM((_CHUNK, 128), jnp.float32),
            pltpu.VMEM_SHARED((_NP, 128), jnp.float32),
            pltpu.SemaphoreType.DMA,
            pltpu.SemaphoreType.DMA,
            pltpu.SemaphoreType.DMA,
            pltpu.SemaphoreType.DMA,
        ],
    )
    def k(*refs):
        tables = refs[:n_tables]
        (src_hbm, dst_hbm, out_hbm, isrc_v, idst_v, rows_a, rows_b, acc_sh,
         sga, sgb, ssa, ssb) = refs[n_tables:]
        c = lax.axis_index("c")
        s = lax.axis_index("s")
        wid = c * _NS + s
        pltpu.sync_copy(src_hbm.at[pl.ds(wid * _ET, _ET)], isrc_v)
        pltpu.sync_copy(dst_hbm.at[wid], idst_v)

        def zfill(i, _):
            for j in range(8):
                rows_a[i, pl.ds(16 * j, 16)] = jnp.zeros((16,), jnp.float32)
            return 0

        for t in range(n_tables):
            tbl = tables[t]
            lax.fori_loop(0, _CHUNK, zfill, 0)
            for j in range(_ROWS_T // _CHUNK):
                pltpu.sync_copy(rows_a, acc_sh.at[pl.ds(s * _ROWS_T + j * _CHUNK, _CHUNK)])
            pltpu.sync_copy(
                rows_a.at[pl.ds(0, _ROWS_T % _CHUNK)],
                acc_sh.at[pl.ds(s * _ROWS_T + (_ROWS_T // _CHUNK) * _CHUNK, _ROWS_T % _CHUNK)],
            )
            plsc.subcore_barrier()

            pltpu.async_copy(tbl.at[isrc_v.at[pl.ds(0, _CHUNK)]], rows_a, sga)
            pltpu.async_copy(tbl.at[isrc_v.at[pl.ds(_CHUNK, _CHUNK)]], rows_b, sgb)

            def body(p, _):
                i0 = 2 * p
                i1 = i0 + 1
                pltpu.make_async_copy(tbl.at[isrc_v.at[pl.ds(i0 * _CHUNK, _CHUNK)]], rows_a, sga).wait()
                pltpu.async_copy(rows_a, acc_sh.at[idst_v.at[i0]], ssa, add=True)
                pltpu.make_async_copy(tbl.at[isrc_v.at[pl.ds(i1 * _CHUNK, _CHUNK)]], rows_b, sgb).wait()
                pltpu.async_copy(rows_b, acc_sh.at[idst_v.at[i1]], ssb, add=True)
                pltpu.make_async_copy(rows_a, acc_sh.at[idst_v.at[i0]], ssa).wait()

                @pl.when(i0 + 2 < _NCHUNK)
                def _():
                    pltpu.async_copy(tbl.at[isrc_v.at[pl.ds((i0 + 2) * _CHUNK, _CHUNK)]], rows_a, sga)

                pltpu.make_async_copy(rows_b, acc_sh.at[idst_v.at[i1]], ssb).wait()

                @pl.when(i1 + 2 < _NCHUNK)
                def _():
                    pltpu.async_copy(tbl.at[isrc_v.at[pl.ds((i1 + 2) * _CHUNK, _CHUNK)]], rows_b, sgb)

                return 0

            lax.fori_loop(0, _NCHUNK // 2, body, 0)
            if _NCHUNK % 2:
                ilast = _NCHUNK - 1
                pltpu.make_async_copy(tbl.at[isrc_v.at[pl.ds(ilast * _CHUNK, _CHUNK)]], rows_a, sga).wait()
                pltpu.async_copy(rows_a, acc_sh.at[idst_v.at[ilast]], ssa, add=True)
                pltpu.make_async_copy(rows_a, acc_sh.at[idst_v.at[ilast]], ssa).wait()
            plsc.subcore_barrier()
            pltpu.sync_copy(
                acc_sh.at[pl.ds(s * _ROWS_T, _ROWS_T)],
                out_hbm.at[t, c, pl.ds(s * _ROWS_T, _ROWS_T)],
            )
            if t + 1 < n_tables:
                plsc.subcore_barrier()

    return k


_scatter1 = _make_scatter(1)


_BLK = 2000
_GRID = _N // _BLK


def _tcA_call(x, deg_p):
    def body(x_ref, dp_ref, z_ref):
        deg = dp_ref[0] + dp_ref[1]
        dinv = lax.rsqrt(deg[:, :1] + 1.0)
        z_ref[:, :] = x_ref[:, :] * dinv

    return pl.pallas_call(
        body,
        grid=(_GRID,),
        in_specs=[
            pl.BlockSpec((_BLK, _IN), lambda i: (i, 0)),
            pl.BlockSpec((_NC, _BLK, _DEG_W), lambda i: (0, i, 0)),
        ],
        out_specs=pl.BlockSpec((_BLK, _IN), lambda i: (i, 0)),
        out_shape=jax.ShapeDtypeStruct((_N, _IN), jnp.float32),
    )(x, deg_p)


def _tcB_call(agg1, x, deg_p, b1_r, W1, W2):
    def body(a_ref, x_ref, dp_ref, b1_ref, w1_ref, w2_ref, y2_ref, s2_ref):
        deg = dp_ref[0] + dp_ref[1]
        dinv = lax.rsqrt(deg[:, :1] + 1.0)
        u = (a_ref[0, 0] + a_ref[0, 1]) * dinv + x_ref[:, :] * (dinv * dinv)
        h = jnp.dot(u, w1_ref[:, :], preferred_element_type=jnp.float32)
        h = jnp.maximum(h + b1_ref[0:1, :], 0.0)
        xw = jnp.dot(h, w2_ref[:, :], preferred_element_type=jnp.float32)
        y2_ref[:, :] = xw * dinv
        s2_ref[:, :] = xw * (dinv * dinv)

    return pl.pallas_call(
        body,
        grid=(_GRID,),
        in_specs=[
            pl.BlockSpec((1, _NC, _BLK, 128), lambda i: (0, 0, i, 0)),
            pl.BlockSpec((_BLK, _IN), lambda i: (i, 0)),
            pl.BlockSpec((_NC, _BLK, _DEG_W), lambda i: (0, i, 0)),
            pl.BlockSpec((1, _HID), lambda i: (0, 0)),
            pl.BlockSpec((_IN, _HID), lambda i: (0, 0)),
            pl.BlockSpec((_HID, _OUT), lambda i: (0, 0)),
        ],
        out_specs=[
            pl.BlockSpec((_BLK, _OUT), lambda i: (i, 0)),
            pl.BlockSpec((_BLK, _OUT), lambda i: (i, 0)),
        ],
        out_shape=[
            jax.ShapeDtypeStruct((_N, _OUT), jnp.float32),
            jax.ShapeDtypeStruct((_N, _OUT), jnp.float32),
        ],
    )(agg1, x, deg_p, b1_r, W1, W2)


def _tcC_call(acc2, self2, deg_p, b2_r):
    def body(a_ref, s2_ref, dp_ref, b2_ref, o_ref):
        deg = dp_ref[0] + dp_ref[1]
        dinv = lax.rsqrt(deg[:, :1] + 1.0)
        acc = a_ref[0, 0] + a_ref[0, 1]
        o_ref[:, :] = acc * dinv + s2_ref[:, :] + b2_ref[0:1, :]

    return pl.pallas_call(
        body,
        grid=(_GRID,),
        in_specs=[
            pl.BlockSpec((1, _NC, _BLK, 128), lambda i: (0, 0, i, 0)),
            pl.BlockSpec((_BLK, _OUT), lambda i: (i, 0)),
            pl.BlockSpec((_NC, _BLK, _DEG_W), lambda i: (0, i, 0)),
            pl.BlockSpec((1, 128), lambda i: (0, 0)),
        ],
        out_specs=pl.BlockSpec((_BLK, _OUT), lambda i: (i, 0)),
        out_shape=jax.ShapeDtypeStruct((_N, _OUT), jnp.float32),
    )(acc2, self2, deg_p, b2_r)


def kernel(x, edge_index, W1, b1, W2, b2):
    src1d = edge_index[0].astype(jnp.int32)
    dst2d = edge_index[1].astype(jnp.int32).reshape(_NW, _NCHUNK, _CHUNK)
    b1_r = b1.reshape(1, _HID)
    b2_r = b2.reshape(1, 128)

    ones_c = jnp.ones((_CHUNK, _DEG_W), jnp.float32)
    zeros_c = jnp.zeros((128, _DEG_W), jnp.float32)
    deg_p = _deg_call(dst2d, ones_c, zeros_c)
    z = _tcA_call(x, deg_p)
    agg1 = _scatter1(z, src1d, dst2d)
    y2, self2 = _tcB_call(agg1, x, deg_p, b1_r, W1, W2)
    acc2 = _scatter1(y2, src1d, dst2d)
    out = _tcC_call(acc2, self2, deg_p, b2_r)
    return out
